# Initial kernel scaffold; baseline (speedup 1.0000x reference)
import jax
import jax.numpy as jnp
from jax.experimental import pallas as pl


def kernel(x, gate_w, W1, W3, W2, sw1, sw3, sw2):
    raise NotImplementedError("write your pallas kernel here")


# speedup vs baseline: 6.1029x; 6.1029x over previous
import functools

import jax
import jax.numpy as jnp
from jax.experimental import pallas as pl
from jax.experimental.pallas import tpu as pltpu

DIM = 2048
INTER = 1024
E = 64

TM = 128
TR = 256


def _router_body(x_ref, gw_ref, eid_ref, g_ref):
    logits = jax.lax.dot_general(
        gw_ref[...], x_ref[...],
        dimension_numbers=(((1,), (1,)), ((), ())),
        preferred_element_type=jnp.float32)
    scores = jax.nn.sigmoid(logits)
    eid = jnp.argmax(scores, axis=0).astype(jnp.int32)
    smax = jnp.max(scores, axis=0)
    g = smax / jnp.maximum(smax, 1e-12)
    eid_ref[0, 0, :] = eid
    g_ref[0, 0, :] = g


def _route(xf, gate_w, t):
    nr = t // TR
    eid3, g3 = pl.pallas_call(
        _router_body,
        grid=(nr,),
        in_specs=[
            pl.BlockSpec((TR, DIM), lambda r: (r, 0)),
            pl.BlockSpec((E, DIM), lambda r: (0, 0)),
        ],
        out_specs=[
            pl.BlockSpec((1, 1, TR), lambda r: (r, 0, 0)),
            pl.BlockSpec((1, 1, TR), lambda r: (r, 0, 0)),
        ],
        out_shape=[
            jax.ShapeDtypeStruct((nr, 1, TR), jnp.int32),
            jax.ShapeDtypeStruct((nr, 1, TR), jnp.float32),
        ],
    )(xf, gate_w)
    return eid3.reshape(t), g3.reshape(t)


def _gmm_body(sm_ref, se_ref, sfl_ref, soff_ref,
              x_ref, w1_ref, w3_ref, w2_ref, sw1_ref, sw3_ref, sw2_ref,
              g_ref, out_ref):
    s = pl.program_id(0)
    e = se_ref[s]
    fl = sfl_ref[s]
    start = soff_ref[e]
    end = soff_ref[e + 1]
    row0 = sm_ref[s] * TM

    x = x_ref[...]

    @pl.when((fl & 2) != 0)
    def _shared():
        h1 = jax.lax.dot_general(
            x, sw1_ref[...], (((1,), (1,)), ((), ())),
            preferred_element_type=jnp.float32)
        h3 = jax.lax.dot_general(
            x, sw3_ref[...], (((1,), (1,)), ((), ())),
            preferred_element_type=jnp.float32)
        hh = (h1 * jax.nn.sigmoid(h1) * h3).astype(jnp.bfloat16)
        o = jax.lax.dot_general(
            hh, sw2_ref[...], (((1,), (1,)), ((), ())),
            preferred_element_type=jnp.float32)
        out_ref[...] = o.astype(jnp.bfloat16)

    @pl.when((fl & 1) != 0)
    def _expert():
        h1 = jnp.dot(x, w1_ref[0], preferred_element_type=jnp.float32)
        h3 = jnp.dot(x, w3_ref[0], preferred_element_type=jnp.float32)
        hh = (h1 * jax.nn.sigmoid(h1) * h3).astype(jnp.bfloat16)
        o = jnp.dot(hh, w2_ref[0], preferred_element_type=jnp.float32)
        rows = row0 + jax.lax.broadcasted_iota(jnp.int32, (TM, 1), 0)
        mask = (rows >= start) & (rows < end)
        gcol = g_ref[0, 0, :].reshape(TM, 1)
        contrib = jnp.where(mask, o * gcol, 0.0)
        out_ref[...] = (out_ref[...].astype(jnp.float32)
                        + contrib).astype(jnp.bfloat16)


def _gmm(xs, w1, w3, w2, sw1, sw3, sw2, g3, sm, se, sfl, soff, t):
    nsteps = sm.shape[0]
    grid_spec = pltpu.PrefetchScalarGridSpec(
        num_scalar_prefetch=4,
        grid=(nsteps,),
        in_specs=[
            pl.BlockSpec((TM, DIM), lambda s, sm, se, sfl, soff: (sm[s], 0)),
            pl.BlockSpec((1, DIM, INTER),
                         lambda s, sm, se, sfl, soff: (se[s], 0, 0)),
            pl.BlockSpec((1, DIM, INTER),
                         lambda s, sm, se, sfl, soff: (se[s], 0, 0)),
            pl.BlockSpec((1, INTER, DIM),
                         lambda s, sm, se, sfl, soff: (se[s], 0, 0)),
            pl.BlockSpec((INTER, DIM), lambda s, sm, se, sfl, soff: (0, 0)),
            pl.BlockSpec((INTER, DIM), lambda s, sm, se, sfl, soff: (0, 0)),
            pl.BlockSpec((DIM, INTER), lambda s, sm, se, sfl, soff: (0, 0)),
            pl.BlockSpec((1, 1, TM),
                         lambda s, sm, se, sfl, soff: (sm[s], 0, 0)),
        ],
        out_specs=pl.BlockSpec((TM, DIM),
                               lambda s, sm, se, sfl, soff: (sm[s], 0)),
    )
    return pl.pallas_call(
        _gmm_body,
        grid_spec=grid_spec,
        out_shape=jax.ShapeDtypeStruct((t, DIM), jnp.bfloat16),
    )(sm, se, sfl, soff, xs, w1, w3, w2, sw1, sw3, sw2, g3)


def _schedule(eid, t):
    ntiles = t // TM
    nsteps = ntiles + E - 1

    counts = jnp.bincount(eid, length=E).astype(jnp.int32)
    ends = jnp.cumsum(counts)
    starts = ends - counts
    soff = jnp.concatenate([jnp.zeros((1,), jnp.int32), ends]).astype(jnp.int32)

    perm = jnp.argsort(eid, stable=True)
    pos = jnp.zeros((t,), jnp.int32).at[perm].set(
        jnp.arange(t, dtype=jnp.int32))

    nonzero = counts > 0
    t_first = starts // TM
    t_last = jnp.where(nonzero, (ends - 1) // TM, 0)
    ntile_e = jnp.where(nonzero, t_last - t_first + 1, 0)
    cume = jnp.cumsum(ntile_e)
    cume_ex = cume - ntile_e
    total = cume[-1]

    sidx = jnp.arange(nsteps, dtype=jnp.int32)
    e_arr = jnp.clip(jnp.searchsorted(cume, sidx, side='right'),
                     0, E - 1).astype(jnp.int32)
    m_arr = (t_first[e_arr] + sidx - cume_ex[e_arr]).astype(jnp.int32)
    active = sidx < total

    e_last = jnp.max(jnp.where(nonzero, jnp.arange(E, dtype=jnp.int32), -1))
    m_arr = jnp.where(active, m_arr, ntiles - 1)
    e_arr = jnp.where(active, e_arr, e_last)

    first = jnp.concatenate(
        [jnp.ones((1,), jnp.bool_), m_arr[1:] != m_arr[:-1]])
    sfl = (active.astype(jnp.int32) + 2 * first.astype(jnp.int32))
    return perm, pos, soff, m_arr, e_arr, sfl


@jax.jit
def kernel(x, gate_w, W1, W3, W2, sw1, sw3, sw2):
    xf = x.reshape(-1, DIM)
    t = xf.shape[0]

    eid, g = _route(xf, gate_w, t)
    perm, pos, soff, sm, se, sfl = _schedule(eid, t)

    xs = jnp.take(xf, perm, axis=0)
    gs = jnp.take(g, perm).astype(jnp.bfloat16).astype(jnp.float32)
    g3 = gs.reshape(t // TM, 1, TM)

    out_sorted = _gmm(xs, W1, W3, W2, sw1, sw3, sw2, g3,
                      sm, se, sfl, soff, t)
    out = jnp.take(out_sorted, pos, axis=0)
    return out.reshape(x.shape)
